# baseline (speedup 1.0000x reference)
import functools

import jax
import jax.numpy as jnp
from jax import lax
from jax.experimental import pallas as pl
from jax.experimental.pallas import tpu as pltpu
from jax.experimental.pallas import tpu_sc as plsc

ROWS = 32
VOCAB = 1_000_000
BLK = 16384
TW = 128
U = 16
SC_BLOCKS = 10
SC_COLS = SC_BLOCKS * BLK
TC_GRID = (VOCAB - SC_COLS + BLK - 1) // BLK
SC_CH = 16384
SC_USC = 4

_TINY = 1.1754943508222875e-38
_BIG_IDX = 2**30

_LOG_COEF = (0.9999999995951486, -0.4999997504803172, 0.3333332364301691,
             -0.2500243000794166, 0.20002637431192238, -0.1659335226848822,
             0.14162240368094145, -0.13335290951259632, 0.13048994247686133,
             -0.07592118758319573)
_LN2_HI = 0.693359375
_LN2_LO = -2.12194440e-4


def _threefry_bits(j):
    rotations = ((13, 15, 26, 6), (17, 29, 16, 24))
    k0 = jnp.uint32(0)
    k1 = jnp.uint32(42)
    ks = (k0, k1, jnp.uint32(0x1BD11BDA) ^ k0 ^ k1)
    x0 = jnp.zeros_like(j) + ks[0]
    x1 = j + ks[1]

    def rotl(x, d):
        return (x << jnp.uint32(d)) | (x >> jnp.uint32(32 - d))

    for i in range(5):
        for r in rotations[i % 2]:
            x0 = x0 + x1
            x1 = rotl(x1, r)
            x1 = x0 ^ x1
        x0 = x0 + ks[(i + 1) % 3]
        x1 = x1 + ks[(i + 2) % 3] + jnp.uint32(i + 1)
    return x0 ^ x1


def _partials_kernel(x_ref, vals_ref, idxs_ref):
    b = pl.program_id(0)
    row_off = lax.broadcasted_iota(jnp.uint32, (ROWS, TW), 0) * jnp.uint32(VOCAB)
    lane = lax.broadcasted_iota(jnp.uint32, (ROWS, TW), 1)
    tiny = jnp.float32(_TINY)

    def body(t, carry):
        acc_max, acc_idx = carry
        for s in range(U):
            base = (SC_COLS + b * BLK + (t * U + s) * TW).astype(jnp.uint32)
            col = lane + base
            j = row_off + col
            bits = _threefry_bits(j)
            ubits = (bits >> jnp.uint32(9)) | jnp.uint32(0x3F800000)
            f = lax.bitcast_convert_type(ubits, jnp.float32) - jnp.float32(1.0)
            u = jnp.maximum(f, tiny)
            g = -jnp.log(-jnp.log(u))
            score = g + x_ref[:, pl.ds((t * U + s) * TW, TW)]
            score = jnp.where(col < jnp.uint32(VOCAB), score, -jnp.inf)
            upd = score > acc_max
            acc_idx = jnp.where(upd, col.astype(jnp.int32), acc_idx)
            acc_max = jnp.maximum(acc_max, score)
        return acc_max, acc_idx

    acc_max0 = jnp.full((ROWS, TW), -jnp.inf, jnp.float32)
    acc_idx0 = jnp.zeros((ROWS, TW), jnp.int32)
    acc_max, acc_idx = lax.fori_loop(0, BLK // (TW * U), body, (acc_max0, acc_idx0))
    vals_ref[...] = acc_max
    idxs_ref[...] = acc_idx


def _alog(v):
    b = lax.bitcast_convert_type(v, jnp.uint32)
    bm = b + jnp.uint32(0x3F800000 - 0x3F3504F3)
    e_i = (bm >> jnp.uint32(23)).astype(jnp.int32) - jnp.int32(127)
    x = lax.bitcast_convert_type(
        b - lax.bitcast_convert_type(e_i << jnp.int32(23), jnp.uint32),
        jnp.float32)
    r = x - jnp.float32(1.0)
    p = jnp.float32(_LOG_COEF[-1])
    for c in _LOG_COEF[-2::-1]:
        p = p * r + jnp.float32(c)
    p = p * r
    ef = e_i.astype(jnp.float32)
    return ef * jnp.float32(_LN2_HI) + (ef * jnp.float32(_LN2_LO) + p)


def _sc_body(x_hbm, vals_hbm, idxs_hbm, buf, val_st, idx_st):
    core = lax.axis_index("c")
    sub = lax.axis_index("s")
    row = sub * 2 + core
    row_off = lax.bitcast_convert_type(row * VOCAB, jnp.uint32)
    iota = lax.iota(jnp.int32, 16)
    tiny = jnp.float32(_TINY)

    def chunk_step(k, carry):
        start = k * SC_CH
        pltpu.sync_copy(x_hbm.at[pl.ds(row * VOCAB + start, SC_CH)], buf)

        def body(t, carry2):
            a_max, a_idx = carry2
            for s in range(SC_USC):
                off = t * (16 * SC_USC) + s * 16
                col = iota + (start + off)
                j = lax.bitcast_convert_type(col, jnp.uint32) + row_off
                bits = _threefry_bits(j)
                ubits = (bits >> jnp.uint32(9)) | jnp.uint32(0x3F800000)
                f = lax.bitcast_convert_type(ubits, jnp.float32) - jnp.float32(1.0)
                u = jnp.maximum(f, tiny)
                g = -_alog(-_alog(u))
                score = g + buf[pl.ds(off, 16)]
                upd = score > a_max
                a_idx = jnp.where(upd, col, a_idx)
                a_max = jnp.maximum(a_max, score)
            return a_max, a_idx

        return lax.fori_loop(0, SC_CH // (16 * SC_USC), body, carry)

    acc_max0 = jnp.full((16,), -jnp.inf, jnp.float32)
    acc_idx0 = jnp.zeros((16,), jnp.int32)
    acc_max, acc_idx = lax.fori_loop(0, SC_COLS // SC_CH, chunk_step,
                                     (acc_max0, acc_idx0))
    val_st[...] = acc_max
    idx_st[...] = acc_idx
    pltpu.sync_copy(val_st, vals_hbm.at[pl.ds(row * 16, 16)])
    pltpu.sync_copy(idx_st, idxs_hbm.at[pl.ds(row * 16, 16)])


_sc_sample = functools.partial(
    pl.kernel,
    out_type=[
        jax.ShapeDtypeStruct((ROWS * 16,), jnp.float32),
        jax.ShapeDtypeStruct((ROWS * 16,), jnp.int32),
    ],
    mesh=plsc.VectorSubcoreMesh(
        core_axis_name="c", subcore_axis_name="s", num_cores=2, num_subcores=16),
    scratch_types=[
        pltpu.VMEM((SC_CH,), jnp.float32),
        pltpu.VMEM((16,), jnp.float32),
        pltpu.VMEM((16,), jnp.int32),
    ],
)(_sc_body)


def _merge_kernel(tv_ref, ti_ref, sv_ref, si_ref, out_ref):
    tv = tv_ref[...]
    ti = ti_ref[...]
    sv = sv_ref[...]
    si = si_ref[...]
    m = jnp.maximum(jnp.max(tv, axis=1, keepdims=True),
                    jnp.max(sv, axis=1, keepdims=True))
    ct = jnp.min(jnp.where(tv == m, ti, jnp.int32(_BIG_IDX)), axis=1, keepdims=True)
    cs = jnp.min(jnp.where(sv == m, si, jnp.int32(_BIG_IDX)), axis=1, keepdims=True)
    out_ref[...] = jnp.minimum(ct, cs)


@jax.jit
def _sample(inputs):
    sc_vals, sc_idxs = _sc_sample(inputs.reshape(-1))
    tc_vals, tc_idxs = pl.pallas_call(
        _partials_kernel,
        grid=(TC_GRID,),
        in_specs=[pl.BlockSpec((ROWS, BLK), lambda b: (0, b + SC_BLOCKS))],
        out_specs=[
            pl.BlockSpec((ROWS, TW), lambda b: (0, b)),
            pl.BlockSpec((ROWS, TW), lambda b: (0, b)),
        ],
        out_shape=[
            jax.ShapeDtypeStruct((ROWS, TC_GRID * TW), jnp.float32),
            jax.ShapeDtypeStruct((ROWS, TC_GRID * TW), jnp.int32),
        ],
        compiler_params=pltpu.CompilerParams(
            dimension_semantics=("parallel",),
        ),
    )(inputs)
    out = pl.pallas_call(
        _merge_kernel,
        out_shape=jax.ShapeDtypeStruct((ROWS, 1), jnp.int32),
    )(tc_vals, tc_idxs, sc_vals.reshape(ROWS, 16), sc_idxs.reshape(ROWS, 16))
    return out.reshape(ROWS)


def kernel(inputs, states):
    predicted_ids = _sample(inputs)
    return (predicted_ids, states)


# speedup vs baseline: 6.2020x; 6.2020x over previous
import functools

import jax
import jax.numpy as jnp
from jax import lax
from jax.experimental import pallas as pl
from jax.experimental.pallas import tpu as pltpu
from jax.experimental.pallas import tpu_sc as plsc

ROWS = 32
VOCAB = 1_000_000
BLK = 16384
TW = 128
U = 16
SC_BLOCKS = 10
SC_COLS = SC_BLOCKS * BLK
TC_GRID = (VOCAB - SC_COLS + BLK - 1) // BLK
SC_CH = 16384
SC_USC = 4

_TINY = 1.1754943508222875e-38
_BIG_IDX = 2**30

_LOG_COEF = (0.9999999995951486, -0.4999997504803172, 0.3333332364301691,
             -0.2500243000794166, 0.20002637431192238, -0.1659335226848822,
             0.14162240368094145, -0.13335290951259632, 0.13048994247686133,
             -0.07592118758319573)
_LN2_HI = 0.693359375
_LN2_LO = -2.12194440e-4


def _threefry_bits(j):
    rotations = ((13, 15, 26, 6), (17, 29, 16, 24))
    k0 = jnp.uint32(0)
    k1 = jnp.uint32(42)
    ks = (k0, k1, jnp.uint32(0x1BD11BDA) ^ k0 ^ k1)
    x0 = jnp.zeros_like(j) + ks[0]
    x1 = j + ks[1]

    def rotl(x, d):
        return (x << jnp.uint32(d)) | (x >> jnp.uint32(32 - d))

    for i in range(5):
        for r in rotations[i % 2]:
            x0 = x0 + x1
            x1 = rotl(x1, r)
            x1 = x0 ^ x1
        x0 = x0 + ks[(i + 1) % 3]
        x1 = x1 + ks[(i + 2) % 3] + jnp.uint32(i + 1)
    return x0 ^ x1


def _partials_kernel(x_ref, vals_ref, idxs_ref):
    b = pl.program_id(0)
    row_off = lax.broadcasted_iota(jnp.uint32, (ROWS, TW), 0) * jnp.uint32(VOCAB)
    lane = lax.broadcasted_iota(jnp.uint32, (ROWS, TW), 1)
    tiny = jnp.float32(_TINY)

    def body(t, carry):
        acc_max, acc_idx = carry
        for s in range(U):
            base = (SC_COLS + b * BLK + (t * U + s) * TW).astype(jnp.uint32)
            col = lane + base
            j = row_off + col
            bits = _threefry_bits(j)
            ubits = (bits >> jnp.uint32(9)) | jnp.uint32(0x3F800000)
            f = lax.bitcast_convert_type(ubits, jnp.float32) - jnp.float32(1.0)
            u = jnp.maximum(f, tiny)
            g = -jnp.log(-jnp.log(u))
            score = g + x_ref[:, pl.ds((t * U + s) * TW, TW)]
            score = jnp.where(col < jnp.uint32(VOCAB), score, -jnp.inf)
            upd = score > acc_max
            acc_idx = jnp.where(upd, col.astype(jnp.int32), acc_idx)
            acc_max = jnp.maximum(acc_max, score)
        return acc_max, acc_idx

    acc_max0 = jnp.full((ROWS, TW), -jnp.inf, jnp.float32)
    acc_idx0 = jnp.zeros((ROWS, TW), jnp.int32)
    acc_max, acc_idx = lax.fori_loop(0, BLK // (TW * U), body, (acc_max0, acc_idx0))
    vals_ref[...] = acc_max
    idxs_ref[...] = acc_idx


def _alog(v):
    b = lax.bitcast_convert_type(v, jnp.uint32)
    bm = b + jnp.uint32(0x3F800000 - 0x3F3504F3)
    e_i = (bm >> jnp.uint32(23)).astype(jnp.int32) - jnp.int32(127)
    x = lax.bitcast_convert_type(
        b - lax.bitcast_convert_type(e_i << jnp.int32(23), jnp.uint32),
        jnp.float32)
    r = x - jnp.float32(1.0)
    p = jnp.float32(_LOG_COEF[-1])
    for c in _LOG_COEF[-2::-1]:
        p = p * r + jnp.float32(c)
    p = p * r
    ef = e_i.astype(jnp.float32)
    return ef * jnp.float32(_LN2_HI) + (ef * jnp.float32(_LN2_LO) + p)


def _sc_body(x_hbm, vals_hbm, idxs_hbm, buf, val_st, idx_st):
    core = lax.axis_index("c")
    sub = lax.axis_index("s")
    row = sub * 2 + core
    row_off = lax.bitcast_convert_type(row * VOCAB, jnp.uint32)
    iota = lax.iota(jnp.int32, 16)
    tiny = jnp.float32(_TINY)

    def chunk_step(k, carry):
        start = k * SC_CH
        pltpu.sync_copy(x_hbm.at[row, pl.ds(start, SC_CH)], buf)

        def body(t, carry2):
            a_max, a_idx = carry2
            for s in range(SC_USC):
                off = t * (16 * SC_USC) + s * 16
                col = iota + (start + off)
                j = lax.bitcast_convert_type(col, jnp.uint32) + row_off
                bits = _threefry_bits(j)
                ubits = (bits >> jnp.uint32(9)) | jnp.uint32(0x3F800000)
                f = lax.bitcast_convert_type(ubits, jnp.float32) - jnp.float32(1.0)
                u = jnp.maximum(f, tiny)
                g = -_alog(-_alog(u))
                score = g + buf[pl.ds(off, 16)]
                upd = score > a_max
                a_idx = jnp.where(upd, col, a_idx)
                a_max = jnp.maximum(a_max, score)
            return a_max, a_idx

        return lax.fori_loop(0, SC_CH // (16 * SC_USC), body, carry)

    acc_max0 = jnp.full((16,), -jnp.inf, jnp.float32)
    acc_idx0 = jnp.zeros((16,), jnp.int32)
    acc_max, acc_idx = lax.fori_loop(0, SC_COLS // SC_CH, chunk_step,
                                     (acc_max0, acc_idx0))
    val_st[...] = acc_max
    idx_st[...] = acc_idx
    pltpu.sync_copy(val_st, vals_hbm.at[pl.ds(row * 16, 16)])
    pltpu.sync_copy(idx_st, idxs_hbm.at[pl.ds(row * 16, 16)])


_sc_sample = functools.partial(
    pl.kernel,
    out_type=[
        jax.ShapeDtypeStruct((ROWS * 16,), jnp.float32),
        jax.ShapeDtypeStruct((ROWS * 16,), jnp.int32),
    ],
    mesh=plsc.VectorSubcoreMesh(
        core_axis_name="c", subcore_axis_name="s", num_cores=2, num_subcores=16),
    scratch_types=[
        pltpu.VMEM((SC_CH,), jnp.float32),
        pltpu.VMEM((16,), jnp.float32),
        pltpu.VMEM((16,), jnp.int32),
    ],
)(_sc_body)


def _merge_kernel(tv_ref, ti_ref, sv_ref, si_ref, out_ref):
    tv = tv_ref[...]
    ti = ti_ref[...]
    sv = sv_ref[...]
    si = si_ref[...]
    m = jnp.maximum(jnp.max(tv, axis=1, keepdims=True),
                    jnp.max(sv, axis=1, keepdims=True))
    ct = jnp.min(jnp.where(tv == m, ti, jnp.int32(_BIG_IDX)), axis=1, keepdims=True)
    cs = jnp.min(jnp.where(sv == m, si, jnp.int32(_BIG_IDX)), axis=1, keepdims=True)
    out_ref[...] = jnp.minimum(ct, cs)


@jax.jit
def _sample(inputs):
    sc_vals, sc_idxs = _sc_sample(inputs)
    tc_vals, tc_idxs = pl.pallas_call(
        _partials_kernel,
        grid=(TC_GRID,),
        in_specs=[pl.BlockSpec((ROWS, BLK), lambda b: (0, b + SC_BLOCKS))],
        out_specs=[
            pl.BlockSpec((ROWS, TW), lambda b: (0, b)),
            pl.BlockSpec((ROWS, TW), lambda b: (0, b)),
        ],
        out_shape=[
            jax.ShapeDtypeStruct((ROWS, TC_GRID * TW), jnp.float32),
            jax.ShapeDtypeStruct((ROWS, TC_GRID * TW), jnp.int32),
        ],
        compiler_params=pltpu.CompilerParams(
            dimension_semantics=("parallel",),
        ),
    )(inputs)
    out = pl.pallas_call(
        _merge_kernel,
        out_shape=jax.ShapeDtypeStruct((ROWS, 1), jnp.int32),
    )(tc_vals, tc_idxs, sc_vals.reshape(ROWS, 16), sc_idxs.reshape(ROWS, 16))
    return out.reshape(ROWS)


def kernel(inputs, states):
    predicted_ids = _sample(inputs)
    return (predicted_ids, states)


# speedup vs baseline: 6.2161x; 1.0023x over previous
import functools

import jax
import jax.numpy as jnp
from jax import lax
from jax.experimental import pallas as pl
from jax.experimental.pallas import tpu as pltpu
from jax.experimental.pallas import tpu_sc as plsc

ROWS = 32
VOCAB = 1_000_000
BLK = 16384
TW = 128
U = 16
SC_BLOCKS = 12
SC_COLS = SC_BLOCKS * BLK
TC_GRID = (VOCAB - SC_COLS + BLK - 1) // BLK
SC_CH = 16384
SC_USC = 8

_TINY = 1.1754943508222875e-38
_BIG_IDX = 2**30

_LOG_COEF = (0.999999817001376, -0.5000066882841885, 0.3333614451212253,
             -0.24959257822755826, 0.19873945069940382, -0.17335310434181253,
             0.1641411761984863, -0.10089627453980135)
_LN2_HI = 0.693359375
_LN2_LO = -2.12194440e-4


def _threefry_bits(j):
    rotations = ((13, 15, 26, 6), (17, 29, 16, 24))
    k0 = jnp.uint32(0)
    k1 = jnp.uint32(42)
    ks = (k0, k1, jnp.uint32(0x1BD11BDA) ^ k0 ^ k1)
    x0 = jnp.zeros_like(j) + ks[0]
    x1 = j + ks[1]

    def rotl(x, d):
        return (x << jnp.uint32(d)) | (x >> jnp.uint32(32 - d))

    for i in range(5):
        for r in rotations[i % 2]:
            x0 = x0 + x1
            x1 = rotl(x1, r)
            x1 = x0 ^ x1
        x0 = x0 + ks[(i + 1) % 3]
        x1 = x1 + ks[(i + 2) % 3] + jnp.uint32(i + 1)
    return x0 ^ x1


def _partials_kernel(x_ref, vals_ref, idxs_ref):
    b = pl.program_id(0)
    row_off = lax.broadcasted_iota(jnp.uint32, (ROWS, TW), 0) * jnp.uint32(VOCAB)
    lane = lax.broadcasted_iota(jnp.uint32, (ROWS, TW), 1)
    tiny = jnp.float32(_TINY)

    def body(t, carry):
        acc_max, acc_idx = carry
        for s in range(U):
            base = (SC_COLS + b * BLK + (t * U + s) * TW).astype(jnp.uint32)
            col = lane + base
            j = row_off + col
            bits = _threefry_bits(j)
            ubits = (bits >> jnp.uint32(9)) | jnp.uint32(0x3F800000)
            f = lax.bitcast_convert_type(ubits, jnp.float32) - jnp.float32(1.0)
            u = jnp.maximum(f, tiny)
            score = x_ref[:, pl.ds((t * U + s) * TW, TW)] - jnp.log(-jnp.log(u))
            score = jnp.where(col < jnp.uint32(VOCAB), score, -jnp.inf)
            upd = score > acc_max
            acc_idx = jnp.where(upd, col.astype(jnp.int32), acc_idx)
            acc_max = jnp.maximum(acc_max, score)
        return acc_max, acc_idx

    acc_max0 = jnp.full((ROWS, TW), -jnp.inf, jnp.float32)
    acc_idx0 = jnp.zeros((ROWS, TW), jnp.int32)
    acc_max, acc_idx = lax.fori_loop(0, BLK // (TW * U), body, (acc_max0, acc_idx0))
    vals_ref[...] = acc_max
    idxs_ref[...] = acc_idx


def _log_poly(v, sign):
    b = lax.bitcast_convert_type(v, jnp.uint32)
    bm = b + jnp.uint32(0x3F800000 - 0x3F3504F3)
    e_i = (bm >> jnp.uint32(23)).astype(jnp.int32) - jnp.int32(127)
    x = lax.bitcast_convert_type(
        b - lax.bitcast_convert_type(e_i << jnp.int32(23), jnp.uint32),
        jnp.float32)
    r = x - jnp.float32(1.0)
    p = jnp.float32(sign * _LOG_COEF[-1])
    for c in _LOG_COEF[-2::-1]:
        p = p * r + jnp.float32(sign * c)
    p = p * r
    ef = e_i.astype(jnp.float32)
    return ef * jnp.float32(sign * _LN2_HI) + (ef * jnp.float32(sign * _LN2_LO) + p)


def _sc_body(x_hbm, vals_hbm, idxs_hbm, buf, val_st, idx_st):
    core = lax.axis_index("c")
    sub = lax.axis_index("s")
    row = sub * 2 + core
    row_off = lax.bitcast_convert_type(row * VOCAB, jnp.uint32)
    iota = lax.iota(jnp.int32, 16)
    tiny = jnp.float32(_TINY)

    def chunk_step(k, carry):
        start = k * SC_CH
        pltpu.sync_copy(x_hbm.at[row, pl.ds(start, SC_CH)], buf)

        def body(t, carry2):
            a_max, a_idx = carry2
            for s in range(SC_USC):
                off = t * (16 * SC_USC) + s * 16
                col = iota + (start + off)
                j = lax.bitcast_convert_type(col, jnp.uint32) + row_off
                bits = _threefry_bits(j)
                ubits = (bits >> jnp.uint32(9)) | jnp.uint32(0x3F800000)
                f = lax.bitcast_convert_type(ubits, jnp.float32) - jnp.float32(1.0)
                u = jnp.maximum(f, tiny)
                score = buf[pl.ds(off, 16)] - _log_poly(_log_poly(u, -1), 1)
                upd = score > a_max
                a_idx = jnp.where(upd, col, a_idx)
                a_max = jnp.maximum(a_max, score)
            return a_max, a_idx

        return lax.fori_loop(0, SC_CH // (16 * SC_USC), body, carry)

    acc_max0 = jnp.full((16,), -jnp.inf, jnp.float32)
    acc_idx0 = jnp.zeros((16,), jnp.int32)
    acc_max, acc_idx = lax.fori_loop(0, SC_COLS // SC_CH, chunk_step,
                                     (acc_max0, acc_idx0))
    val_st[...] = acc_max
    idx_st[...] = acc_idx
    pltpu.sync_copy(val_st, vals_hbm.at[pl.ds(row * 16, 16)])
    pltpu.sync_copy(idx_st, idxs_hbm.at[pl.ds(row * 16, 16)])


_sc_sample = functools.partial(
    pl.kernel,
    out_type=[
        jax.ShapeDtypeStruct((ROWS * 16,), jnp.float32),
        jax.ShapeDtypeStruct((ROWS * 16,), jnp.int32),
    ],
    mesh=plsc.VectorSubcoreMesh(
        core_axis_name="c", subcore_axis_name="s", num_cores=2, num_subcores=16),
    scratch_types=[
        pltpu.VMEM((SC_CH,), jnp.float32),
        pltpu.VMEM((16,), jnp.float32),
        pltpu.VMEM((16,), jnp.int32),
    ],
)(_sc_body)


def _merge_kernel(tv_ref, ti_ref, sv_ref, si_ref, out_ref):
    tv = tv_ref[...]
    ti = ti_ref[...]
    sv = sv_ref[...]
    si = si_ref[...]
    m = jnp.maximum(jnp.max(tv, axis=1, keepdims=True),
                    jnp.max(sv, axis=1, keepdims=True))
    ct = jnp.min(jnp.where(tv == m, ti, jnp.int32(_BIG_IDX)), axis=1, keepdims=True)
    cs = jnp.min(jnp.where(sv == m, si, jnp.int32(_BIG_IDX)), axis=1, keepdims=True)
    out_ref[...] = jnp.minimum(ct, cs)


@jax.jit
def _sample(inputs):
    sc_vals, sc_idxs = _sc_sample(inputs)
    tc_vals, tc_idxs = pl.pallas_call(
        _partials_kernel,
        grid=(TC_GRID,),
        in_specs=[pl.BlockSpec((ROWS, BLK), lambda b: (0, b + SC_BLOCKS))],
        out_specs=[
            pl.BlockSpec((ROWS, TW), lambda b: (0, b)),
            pl.BlockSpec((ROWS, TW), lambda b: (0, b)),
        ],
        out_shape=[
            jax.ShapeDtypeStruct((ROWS, TC_GRID * TW), jnp.float32),
            jax.ShapeDtypeStruct((ROWS, TC_GRID * TW), jnp.int32),
        ],
        compiler_params=pltpu.CompilerParams(
            dimension_semantics=("parallel",),
        ),
    )(inputs)
    out = pl.pallas_call(
        _merge_kernel,
        out_shape=jax.ShapeDtypeStruct((ROWS, 1), jnp.int32),
    )(tc_vals, tc_idxs, sc_vals.reshape(ROWS, 16), sc_idxs.reshape(ROWS, 16))
    return out.reshape(ROWS)


def kernel(inputs, states):
    predicted_ids = _sample(inputs)
    return (predicted_ids, states)


# speedup vs baseline: 6.5113x; 1.0475x over previous
import functools

import jax
import jax.numpy as jnp
from jax import lax
from jax.experimental import pallas as pl
from jax.experimental.pallas import tpu as pltpu
from jax.experimental.pallas import tpu_sc as plsc

ROWS = 32
VOCAB = 1_000_000
BLK = 16384
TW = 128
U = 16
SC_BLOCKS = 11
SC_COLS = SC_BLOCKS * BLK
FULL_BLOCKS = VOCAB // BLK
TC_GRID = FULL_BLOCKS - SC_BLOCKS
TAIL_START = FULL_BLOCKS * BLK
TAIL_COLS = 512
SLIVER_START = TAIL_START + TAIL_COLS
SLIVER_COLS = VOCAB - SLIVER_START
SC_CH = 16384
SC_USC = 8
SC_USC_TAIL = 4

_TINY = 1.1754943508222875e-38
_BIG_IDX = 2**30

_LOG_COEF = (0.999999817001376, -0.5000066882841885, 0.3333614451212253,
             -0.24959257822755826, 0.19873945069940382, -0.17335310434181253,
             0.1641411761984863, -0.10089627453980135)
_LN2_HI = 0.693359375
_LN2_LO = -2.12194440e-4


def _threefry_bits(j):
    rotations = ((13, 15, 26, 6), (17, 29, 16, 24))
    k0 = jnp.uint32(0)
    k1 = jnp.uint32(42)
    ks = (k0, k1, jnp.uint32(0x1BD11BDA) ^ k0 ^ k1)
    x0 = jnp.zeros_like(j) + ks[0]
    x1 = j + ks[1]

    def rotl(x, d):
        return (x << jnp.uint32(d)) | (x >> jnp.uint32(32 - d))

    for i in range(5):
        for r in rotations[i % 2]:
            x0 = x0 + x1
            x1 = rotl(x1, r)
            x1 = x0 ^ x1
        x0 = x0 + ks[(i + 1) % 3]
        x1 = x1 + ks[(i + 2) % 3] + jnp.uint32(i + 1)
    return x0 ^ x1


def _partials_kernel(x_ref, vals_ref, idxs_ref):
    b = pl.program_id(0)
    row_off = lax.broadcasted_iota(jnp.uint32, (ROWS, TW), 0) * jnp.uint32(VOCAB)
    lane = lax.broadcasted_iota(jnp.uint32, (ROWS, TW), 1)
    joff = row_off + lane
    tiny = jnp.float32(_TINY)

    def body(t, carry):
        acc_max, acc_j = carry
        for s in range(U):
            base = (SC_COLS + b * BLK + (t * U + s) * TW).astype(jnp.uint32)
            j = joff + base
            bits = _threefry_bits(j)
            ubits = (bits >> jnp.uint32(9)) | jnp.uint32(0x3F800000)
            f = lax.bitcast_convert_type(ubits, jnp.float32) - jnp.float32(1.0)
            u = jnp.maximum(f, tiny)
            score = x_ref[:, pl.ds((t * U + s) * TW, TW)] - jnp.log(-jnp.log(u))
            upd = score > acc_max
            acc_j = jnp.where(upd, j.astype(jnp.int32), acc_j)
            acc_max = jnp.maximum(acc_max, score)
        return acc_max, acc_j

    acc_max0 = jnp.full((ROWS, TW), -jnp.inf, jnp.float32)
    acc_j0 = jnp.zeros((ROWS, TW), jnp.int32)
    acc_max, acc_j = lax.fori_loop(0, BLK // (TW * U), body, (acc_max0, acc_j0))
    vals_ref[...] = acc_max
    idxs_ref[...] = acc_j - row_off.astype(jnp.int32)


def _log_poly(v, sign):
    b = lax.bitcast_convert_type(v, jnp.uint32)
    bm = b + jnp.uint32(0x3F800000 - 0x3F3504F3)
    e_i = (bm >> jnp.uint32(23)).astype(jnp.int32) - jnp.int32(127)
    x = lax.bitcast_convert_type(
        b - lax.bitcast_convert_type(e_i << jnp.int32(23), jnp.uint32),
        jnp.float32)
    r = x - jnp.float32(1.0)
    p = jnp.float32(sign * _LOG_COEF[-1])
    for c in _LOG_COEF[-2::-1]:
        p = p * r + jnp.float32(sign * c)
    p = p * r
    ef = e_i.astype(jnp.float32)
    return ef * jnp.float32(sign * _LN2_HI) + (ef * jnp.float32(sign * _LN2_LO) + p)


def _sc_body(x_hbm, vals_hbm, idxs_hbm, buf, val_st, idx_st):
    core = lax.axis_index("c")
    sub = lax.axis_index("s")
    row = sub * 2 + core
    row_off = lax.bitcast_convert_type(row * VOCAB, jnp.uint32)
    iota = lax.iota(jnp.int32, 16)
    tiny = jnp.float32(_TINY)

    def scan_buf(start, n_cols, usc, carry):
        def body(t, carry2):
            a_max, a_idx = carry2
            for s in range(usc):
                off = t * (16 * usc) + s * 16
                col = iota + (start + off)
                j = lax.bitcast_convert_type(col, jnp.uint32) + row_off
                bits = _threefry_bits(j)
                ubits = (bits >> jnp.uint32(9)) | jnp.uint32(0x3F800000)
                f = lax.bitcast_convert_type(ubits, jnp.float32) - jnp.float32(1.0)
                u = jnp.maximum(f, tiny)
                score = buf[pl.ds(off, 16)] - _log_poly(_log_poly(u, -1), 1)
                upd = score > a_max
                a_idx = jnp.where(upd, col, a_idx)
                a_max = jnp.maximum(a_max, score)
            return a_max, a_idx

        return lax.fori_loop(0, n_cols // (16 * usc), body, carry)

    def chunk_step(k, carry):
        start = k * SC_CH
        pltpu.sync_copy(x_hbm.at[row, pl.ds(start, SC_CH)], buf)
        return scan_buf(start, SC_CH, SC_USC, carry)

    acc_max0 = jnp.full((16,), -jnp.inf, jnp.float32)
    acc_idx0 = jnp.zeros((16,), jnp.int32)
    carry = lax.fori_loop(0, SC_COLS // SC_CH, chunk_step, (acc_max0, acc_idx0))
    pltpu.sync_copy(x_hbm.at[row, pl.ds(TAIL_START, TAIL_COLS)],
                    buf.at[pl.ds(0, TAIL_COLS)])
    acc_max, acc_idx = scan_buf(TAIL_START, TAIL_COLS, SC_USC_TAIL, carry)
    val_st[...] = acc_max
    idx_st[...] = acc_idx
    pltpu.sync_copy(val_st, vals_hbm.at[pl.ds(row * 16, 16)])
    pltpu.sync_copy(idx_st, idxs_hbm.at[pl.ds(row * 16, 16)])


_sc_sample = functools.partial(
    pl.kernel,
    out_type=[
        jax.ShapeDtypeStruct((ROWS * 16,), jnp.float32),
        jax.ShapeDtypeStruct((ROWS * 16,), jnp.int32),
    ],
    mesh=plsc.VectorSubcoreMesh(
        core_axis_name="c", subcore_axis_name="s", num_cores=2, num_subcores=16),
    scratch_types=[
        pltpu.VMEM((SC_CH,), jnp.float32),
        pltpu.VMEM((16,), jnp.float32),
        pltpu.VMEM((16,), jnp.int32),
    ],
)(_sc_body)


def _merge_kernel(tv_ref, ti_ref, sv_ref, si_ref, xs_ref, out_ref):
    tv = tv_ref[...]
    ti = ti_ref[...]
    sv = sv_ref[...]
    si = si_ref[...]
    row_off = lax.broadcasted_iota(jnp.uint32, (ROWS, SLIVER_COLS), 0) \
        * jnp.uint32(VOCAB)
    lane = lax.broadcasted_iota(jnp.uint32, (ROWS, SLIVER_COLS), 1)
    j = row_off + lane + jnp.uint32(SLIVER_START)
    bits = _threefry_bits(j)
    ubits = (bits >> jnp.uint32(9)) | jnp.uint32(0x3F800000)
    f = lax.bitcast_convert_type(ubits, jnp.float32) - jnp.float32(1.0)
    u = jnp.maximum(f, jnp.float32(_TINY))
    lv = xs_ref[...] - jnp.log(-jnp.log(u))
    li = (lane + jnp.uint32(SLIVER_START)).astype(jnp.int32)

    m = jnp.maximum(jnp.maximum(jnp.max(tv, axis=1, keepdims=True),
                                jnp.max(sv, axis=1, keepdims=True)),
                    jnp.max(lv, axis=1, keepdims=True))
    ct = jnp.min(jnp.where(tv == m, ti, jnp.int32(_BIG_IDX)), axis=1, keepdims=True)
    cs = jnp.min(jnp.where(sv == m, si, jnp.int32(_BIG_IDX)), axis=1, keepdims=True)
    cl = jnp.min(jnp.where(lv == m, li, jnp.int32(_BIG_IDX)), axis=1, keepdims=True)
    out_ref[...] = jnp.minimum(jnp.minimum(ct, cs), cl)


@jax.jit
def _sample(inputs):
    sc_vals, sc_idxs = _sc_sample(inputs)
    tc_vals, tc_idxs = pl.pallas_call(
        _partials_kernel,
        grid=(TC_GRID,),
        in_specs=[pl.BlockSpec((ROWS, BLK), lambda b: (0, b + SC_BLOCKS))],
        out_specs=[
            pl.BlockSpec((ROWS, TW), lambda b: (0, b)),
            pl.BlockSpec((ROWS, TW), lambda b: (0, b)),
        ],
        out_shape=[
            jax.ShapeDtypeStruct((ROWS, TC_GRID * TW), jnp.float32),
            jax.ShapeDtypeStruct((ROWS, TC_GRID * TW), jnp.int32),
        ],
        compiler_params=pltpu.CompilerParams(
            dimension_semantics=("parallel",),
        ),
    )(inputs)
    out = pl.pallas_call(
        _merge_kernel,
        out_shape=jax.ShapeDtypeStruct((ROWS, 1), jnp.int32),
    )(tc_vals, tc_idxs, sc_vals.reshape(ROWS, 16), sc_idxs.reshape(ROWS, 16),
      inputs[:, SLIVER_START:])
    return out.reshape(ROWS)


def kernel(inputs, states):
    predicted_ids = _sample(inputs)
    return (predicted_ids, states)


# speedup vs baseline: 6.5115x; 1.0000x over previous
import functools

import jax
import jax.numpy as jnp
from jax import lax
from jax.experimental import pallas as pl
from jax.experimental.pallas import tpu as pltpu
from jax.experimental.pallas import tpu_sc as plsc

ROWS = 32
VOCAB = 1_000_000
BLK = 16384
TW = 128
U = 16
SC_BLOCKS = 11
SC_COLS = SC_BLOCKS * BLK
FULL_BLOCKS = VOCAB // BLK
TC_GRID = FULL_BLOCKS - SC_BLOCKS
TAIL_START = FULL_BLOCKS * BLK
TAIL_COLS = 512
SLIVER_START = TAIL_START + TAIL_COLS
SLIVER_COLS = VOCAB - SLIVER_START
SC_CH = 16384
SC_USC = 8
SC_USC_TAIL = 4

_TINY = 1.1754943508222875e-38
_BIG_IDX = 2**30

_LOG_COEF = (0.999999817001376, -0.5000066882841885, 0.3333614451212253,
             -0.24959257822755826, 0.19873945069940382, -0.17335310434181253,
             0.1641411761984863, -0.10089627453980135)
_LN2_HI = 0.693359375
_LN2_LO = -2.12194440e-4


def _threefry_bits(j):
    rotations = ((13, 15, 26, 6), (17, 29, 16, 24))
    k0 = jnp.uint32(0)
    k1 = jnp.uint32(42)
    ks = (k0, k1, jnp.uint32(0x1BD11BDA) ^ k0 ^ k1)
    x0 = jnp.zeros_like(j) + ks[0]
    x1 = j + ks[1]

    def rotl(x, d):
        return (x << jnp.uint32(d)) | (x >> jnp.uint32(32 - d))

    for i in range(5):
        for r in rotations[i % 2]:
            x0 = x0 + x1
            x1 = rotl(x1, r)
            x1 = x0 ^ x1
        x0 = x0 + ks[(i + 1) % 3]
        x1 = x1 + ks[(i + 2) % 3] + jnp.uint32(i + 1)
    return x0 ^ x1


def _partials_kernel(x_ref, vals_ref, idxs_ref):
    b = pl.program_id(0)
    row_off = lax.broadcasted_iota(jnp.uint32, (ROWS, TW), 0) * jnp.uint32(VOCAB)
    lane = lax.broadcasted_iota(jnp.uint32, (ROWS, TW), 1)
    joff = row_off + lane
    tiny = jnp.float32(_TINY)

    def body(t, carry):
        acc_max, acc_j = carry
        for s in range(U):
            base = (SC_COLS + b * BLK + (t * U + s) * TW).astype(jnp.uint32)
            j = joff + base
            bits = _threefry_bits(j)
            ubits = (bits >> jnp.uint32(9)) | jnp.uint32(0x3F800000)
            f = lax.bitcast_convert_type(ubits, jnp.float32) - jnp.float32(1.0)
            u = jnp.maximum(f, tiny)
            score = x_ref[:, pl.ds((t * U + s) * TW, TW)] - jnp.log(-jnp.log(u))
            upd = score > acc_max
            acc_j = jnp.where(upd, j.astype(jnp.int32), acc_j)
            acc_max = jnp.maximum(acc_max, score)
        return acc_max, acc_j

    acc_max0 = jnp.full((ROWS, TW), -jnp.inf, jnp.float32)
    acc_j0 = jnp.zeros((ROWS, TW), jnp.int32)
    acc_max, acc_j = lax.fori_loop(0, BLK // (TW * U), body, (acc_max0, acc_j0))
    vals_ref[...] = acc_max
    idxs_ref[...] = acc_j - row_off.astype(jnp.int32)


def _log_poly(v, sign):
    b = lax.bitcast_convert_type(v, jnp.uint32)
    bm = b + jnp.uint32(0x3F800000 - 0x3F3504F3)
    e_i = (bm >> jnp.uint32(23)).astype(jnp.int32) - jnp.int32(127)
    x = lax.bitcast_convert_type(
        b - lax.bitcast_convert_type(e_i << jnp.int32(23), jnp.uint32),
        jnp.float32)
    r = x - jnp.float32(1.0)
    p = jnp.float32(sign * _LOG_COEF[-1])
    for c in _LOG_COEF[-2::-1]:
        p = p * r + jnp.float32(sign * c)
    p = p * r
    ef = e_i.astype(jnp.float32)
    return ef * jnp.float32(sign * _LN2_HI) + (ef * jnp.float32(sign * _LN2_LO) + p)


def _sc_body(x_hbm, vals_hbm, idxs_hbm, buf, val_st, idx_st):
    core = lax.axis_index("c")
    sub = lax.axis_index("s")
    row = sub * 2 + core
    row_off = lax.bitcast_convert_type(row * VOCAB, jnp.uint32)
    iota = lax.iota(jnp.int32, 16)
    tiny = jnp.float32(_TINY)

    def scan_buf(start, n_cols, usc, carry):
        @plsc.parallel_loop(0, n_cols // (16 * usc), carry=carry)
        def body(t, carry2):
            a_max, a_idx = carry2
            for s in range(usc):
                off = t * (16 * usc) + s * 16
                col = iota + (start + off)
                j = lax.bitcast_convert_type(col, jnp.uint32) + row_off
                bits = _threefry_bits(j)
                ubits = (bits >> jnp.uint32(9)) | jnp.uint32(0x3F800000)
                f = lax.bitcast_convert_type(ubits, jnp.float32) - jnp.float32(1.0)
                u = jnp.maximum(f, tiny)
                score = buf[pl.ds(off, 16)] - _log_poly(_log_poly(u, -1), 1)
                upd = score > a_max
                a_idx = jnp.where(upd, col, a_idx)
                a_max = jnp.maximum(a_max, score)
            return a_max, a_idx

        return body

    def chunk_step(k, carry):
        start = k * SC_CH
        pltpu.sync_copy(x_hbm.at[row, pl.ds(start, SC_CH)], buf)
        return scan_buf(start, SC_CH, SC_USC, carry)

    acc_max0 = jnp.full((16,), -jnp.inf, jnp.float32)
    acc_idx0 = jnp.zeros((16,), jnp.int32)
    carry = lax.fori_loop(0, SC_COLS // SC_CH, chunk_step, (acc_max0, acc_idx0))
    pltpu.sync_copy(x_hbm.at[row, pl.ds(TAIL_START, TAIL_COLS)],
                    buf.at[pl.ds(0, TAIL_COLS)])
    acc_max, acc_idx = scan_buf(TAIL_START, TAIL_COLS, SC_USC_TAIL, carry)
    val_st[...] = acc_max
    idx_st[...] = acc_idx
    pltpu.sync_copy(val_st, vals_hbm.at[pl.ds(row * 16, 16)])
    pltpu.sync_copy(idx_st, idxs_hbm.at[pl.ds(row * 16, 16)])


_sc_sample = functools.partial(
    pl.kernel,
    out_type=[
        jax.ShapeDtypeStruct((ROWS * 16,), jnp.float32),
        jax.ShapeDtypeStruct((ROWS * 16,), jnp.int32),
    ],
    mesh=plsc.VectorSubcoreMesh(
        core_axis_name="c", subcore_axis_name="s", num_cores=2, num_subcores=16),
    scratch_types=[
        pltpu.VMEM((SC_CH,), jnp.float32),
        pltpu.VMEM((16,), jnp.float32),
        pltpu.VMEM((16,), jnp.int32),
    ],
)(_sc_body)


def _merge_kernel(tv_ref, ti_ref, sv_ref, si_ref, xs_ref, out_ref):
    tv = tv_ref[...]
    ti = ti_ref[...]
    sv = sv_ref[...]
    si = si_ref[...]
    row_off = lax.broadcasted_iota(jnp.uint32, (ROWS, SLIVER_COLS), 0) \
        * jnp.uint32(VOCAB)
    lane = lax.broadcasted_iota(jnp.uint32, (ROWS, SLIVER_COLS), 1)
    j = row_off + lane + jnp.uint32(SLIVER_START)
    bits = _threefry_bits(j)
    ubits = (bits >> jnp.uint32(9)) | jnp.uint32(0x3F800000)
    f = lax.bitcast_convert_type(ubits, jnp.float32) - jnp.float32(1.0)
    u = jnp.maximum(f, jnp.float32(_TINY))
    lv = xs_ref[...] - jnp.log(-jnp.log(u))
    li = (lane + jnp.uint32(SLIVER_START)).astype(jnp.int32)

    m = jnp.maximum(jnp.maximum(jnp.max(tv, axis=1, keepdims=True),
                                jnp.max(sv, axis=1, keepdims=True)),
                    jnp.max(lv, axis=1, keepdims=True))
    ct = jnp.min(jnp.where(tv == m, ti, jnp.int32(_BIG_IDX)), axis=1, keepdims=True)
    cs = jnp.min(jnp.where(sv == m, si, jnp.int32(_BIG_IDX)), axis=1, keepdims=True)
    cl = jnp.min(jnp.where(lv == m, li, jnp.int32(_BIG_IDX)), axis=1, keepdims=True)
    out_ref[...] = jnp.minimum(jnp.minimum(ct, cs), cl)


@jax.jit
def _sample(inputs):
    sc_vals, sc_idxs = _sc_sample(inputs)
    tc_vals, tc_idxs = pl.pallas_call(
        _partials_kernel,
        grid=(TC_GRID,),
        in_specs=[pl.BlockSpec((ROWS, BLK), lambda b: (0, b + SC_BLOCKS))],
        out_specs=[
            pl.BlockSpec((ROWS, TW), lambda b: (0, b)),
            pl.BlockSpec((ROWS, TW), lambda b: (0, b)),
        ],
        out_shape=[
            jax.ShapeDtypeStruct((ROWS, TC_GRID * TW), jnp.float32),
            jax.ShapeDtypeStruct((ROWS, TC_GRID * TW), jnp.int32),
        ],
        compiler_params=pltpu.CompilerParams(
            dimension_semantics=("parallel",),
        ),
    )(inputs)
    out = pl.pallas_call(
        _merge_kernel,
        out_shape=jax.ShapeDtypeStruct((ROWS, 1), jnp.int32),
    )(tc_vals, tc_idxs, sc_vals.reshape(ROWS, 16), sc_idxs.reshape(ROWS, 16),
      inputs[:, SLIVER_START:])
    return out.reshape(ROWS)


def kernel(inputs, states):
    predicted_ids = _sample(inputs)
    return (predicted_ids, states)


# speedup vs baseline: 6.5595x; 1.0074x over previous
import functools

import jax
import jax.numpy as jnp
from jax import lax
from jax.experimental import pallas as pl
from jax.experimental.pallas import tpu as pltpu
from jax.experimental.pallas import tpu_sc as plsc

ROWS = 32
VOCAB = 1_000_000
BLK = 16384
TW = 128
U = 32
SC_BLOCKS = 11
SC_COLS = SC_BLOCKS * BLK
FULL_BLOCKS = VOCAB // BLK
TC_GRID = FULL_BLOCKS - SC_BLOCKS
TAIL_START = FULL_BLOCKS * BLK
TAIL_COLS = 512
SLIVER_START = TAIL_START + TAIL_COLS
SLIVER_COLS = VOCAB - SLIVER_START
SC_CH = 16384
SC_USC = 8
SC_USC_TAIL = 4

_TINY = 1.1754943508222875e-38
_BIG_IDX = 2**30

_LOG_COEF = (0.999999817001376, -0.5000066882841885, 0.3333614451212253,
             -0.24959257822755826, 0.19873945069940382, -0.17335310434181253,
             0.1641411761984863, -0.10089627453980135)
_LN2_HI = 0.693359375
_LN2_LO = -2.12194440e-4


def _threefry_bits(j):
    rotations = ((13, 15, 26, 6), (17, 29, 16, 24))
    k0 = jnp.uint32(0)
    k1 = jnp.uint32(42)
    ks = (k0, k1, jnp.uint32(0x1BD11BDA) ^ k0 ^ k1)
    x0 = jnp.zeros_like(j) + ks[0]
    x1 = j + ks[1]

    def rotl(x, d):
        return (x << jnp.uint32(d)) | (x >> jnp.uint32(32 - d))

    for i in range(5):
        for r in rotations[i % 2]:
            x0 = x0 + x1
            x1 = rotl(x1, r)
            x1 = x0 ^ x1
        x0 = x0 + ks[(i + 1) % 3]
        x1 = x1 + ks[(i + 2) % 3] + jnp.uint32(i + 1)
    return x0 ^ x1


def _partials_kernel(x_ref, vals_ref, idxs_ref):
    b = pl.program_id(0)
    row_off = lax.broadcasted_iota(jnp.uint32, (ROWS, TW), 0) * jnp.uint32(VOCAB)
    lane = lax.broadcasted_iota(jnp.uint32, (ROWS, TW), 1)
    joff = row_off + lane
    tiny = jnp.float32(_TINY)

    def body(t, carry):
        acc_max, acc_j = carry
        for s in range(U):
            base = (SC_COLS + b * BLK + (t * U + s) * TW).astype(jnp.uint32)
            j = joff + base
            bits = _threefry_bits(j)
            ubits = (bits >> jnp.uint32(9)) | jnp.uint32(0x3F800000)
            f = lax.bitcast_convert_type(ubits, jnp.float32) - jnp.float32(1.0)
            u = jnp.maximum(f, tiny)
            score = x_ref[:, pl.ds((t * U + s) * TW, TW)] - jnp.log(-jnp.log(u))
            upd = score > acc_max
            acc_j = jnp.where(upd, j.astype(jnp.int32), acc_j)
            acc_max = jnp.maximum(acc_max, score)
        return acc_max, acc_j

    acc_max0 = jnp.full((ROWS, TW), -jnp.inf, jnp.float32)
    acc_j0 = jnp.zeros((ROWS, TW), jnp.int32)
    acc_max, acc_j = lax.fori_loop(0, BLK // (TW * U), body, (acc_max0, acc_j0))
    vals_ref[...] = acc_max
    idxs_ref[...] = acc_j - row_off.astype(jnp.int32)


def _log_poly(v, sign):
    b = lax.bitcast_convert_type(v, jnp.uint32)
    bm = b + jnp.uint32(0x3F800000 - 0x3F3504F3)
    e_i = (bm >> jnp.uint32(23)).astype(jnp.int32) - jnp.int32(127)
    x = lax.bitcast_convert_type(
        b - lax.bitcast_convert_type(e_i << jnp.int32(23), jnp.uint32),
        jnp.float32)
    r = x - jnp.float32(1.0)
    p = jnp.float32(sign * _LOG_COEF[-1])
    for c in _LOG_COEF[-2::-1]:
        p = p * r + jnp.float32(sign * c)
    p = p * r
    ef = e_i.astype(jnp.float32)
    return ef * jnp.float32(sign * _LN2_HI) + (ef * jnp.float32(sign * _LN2_LO) + p)


def _sc_body(x_hbm, vals_hbm, idxs_hbm, buf, val_st, idx_st):
    core = lax.axis_index("c")
    sub = lax.axis_index("s")
    row = sub * 2 + core
    row_off = lax.bitcast_convert_type(row * VOCAB, jnp.uint32)
    iota = lax.iota(jnp.int32, 16)
    tiny = jnp.float32(_TINY)

    def scan_buf(start, n_cols, usc, carry):
        @plsc.parallel_loop(0, n_cols // (16 * usc), carry=carry)
        def body(t, carry2):
            a_max, a_idx = carry2
            for s in range(usc):
                off = t * (16 * usc) + s * 16
                col = iota + (start + off)
                j = lax.bitcast_convert_type(col, jnp.uint32) + row_off
                bits = _threefry_bits(j)
                ubits = (bits >> jnp.uint32(9)) | jnp.uint32(0x3F800000)
                f = lax.bitcast_convert_type(ubits, jnp.float32) - jnp.float32(1.0)
                u = jnp.maximum(f, tiny)
                score = buf[pl.ds(off, 16)] - _log_poly(_log_poly(u, -1), 1)
                upd = score > a_max
                a_idx = jnp.where(upd, col, a_idx)
                a_max = jnp.maximum(a_max, score)
            return a_max, a_idx

        return body

    def chunk_step(k, carry):
        start = k * SC_CH
        pltpu.sync_copy(x_hbm.at[row, pl.ds(start, SC_CH)], buf)
        return scan_buf(start, SC_CH, SC_USC, carry)

    acc_max0 = jnp.full((16,), -jnp.inf, jnp.float32)
    acc_idx0 = jnp.zeros((16,), jnp.int32)
    carry = lax.fori_loop(0, SC_COLS // SC_CH, chunk_step, (acc_max0, acc_idx0))
    pltpu.sync_copy(x_hbm.at[row, pl.ds(TAIL_START, TAIL_COLS)],
                    buf.at[pl.ds(0, TAIL_COLS)])
    acc_max, acc_idx = scan_buf(TAIL_START, TAIL_COLS, SC_USC_TAIL, carry)
    val_st[...] = acc_max
    idx_st[...] = acc_idx
    pltpu.sync_copy(val_st, vals_hbm.at[pl.ds(row * 16, 16)])
    pltpu.sync_copy(idx_st, idxs_hbm.at[pl.ds(row * 16, 16)])


_sc_sample = functools.partial(
    pl.kernel,
    out_type=[
        jax.ShapeDtypeStruct((ROWS * 16,), jnp.float32),
        jax.ShapeDtypeStruct((ROWS * 16,), jnp.int32),
    ],
    mesh=plsc.VectorSubcoreMesh(
        core_axis_name="c", subcore_axis_name="s", num_cores=2, num_subcores=16),
    scratch_types=[
        pltpu.VMEM((SC_CH,), jnp.float32),
        pltpu.VMEM((16,), jnp.float32),
        pltpu.VMEM((16,), jnp.int32),
    ],
)(_sc_body)


def _merge_kernel(tv_ref, ti_ref, sv_ref, si_ref, xs_ref, out_ref):
    tv = tv_ref[...]
    ti = ti_ref[...]
    sv = sv_ref[...]
    si = si_ref[...]
    row_off = lax.broadcasted_iota(jnp.uint32, (ROWS, SLIVER_COLS), 0) \
        * jnp.uint32(VOCAB)
    lane = lax.broadcasted_iota(jnp.uint32, (ROWS, SLIVER_COLS), 1)
    j = row_off + lane + jnp.uint32(SLIVER_START)
    bits = _threefry_bits(j)
    ubits = (bits >> jnp.uint32(9)) | jnp.uint32(0x3F800000)
    f = lax.bitcast_convert_type(ubits, jnp.float32) - jnp.float32(1.0)
    u = jnp.maximum(f, jnp.float32(_TINY))
    lv = xs_ref[...] - jnp.log(-jnp.log(u))
    li = (lane + jnp.uint32(SLIVER_START)).astype(jnp.int32)

    m = jnp.maximum(jnp.maximum(jnp.max(tv, axis=1, keepdims=True),
                                jnp.max(sv, axis=1, keepdims=True)),
                    jnp.max(lv, axis=1, keepdims=True))
    ct = jnp.min(jnp.where(tv == m, ti, jnp.int32(_BIG_IDX)), axis=1, keepdims=True)
    cs = jnp.min(jnp.where(sv == m, si, jnp.int32(_BIG_IDX)), axis=1, keepdims=True)
    cl = jnp.min(jnp.where(lv == m, li, jnp.int32(_BIG_IDX)), axis=1, keepdims=True)
    out_ref[...] = jnp.minimum(jnp.minimum(ct, cs), cl)


@jax.jit
def _sample(inputs):
    sc_vals, sc_idxs = _sc_sample(inputs)
    tc_vals, tc_idxs = pl.pallas_call(
        _partials_kernel,
        grid=(TC_GRID,),
        in_specs=[pl.BlockSpec((ROWS, BLK), lambda b: (0, b + SC_BLOCKS))],
        out_specs=[
            pl.BlockSpec((ROWS, TW), lambda b: (0, b)),
            pl.BlockSpec((ROWS, TW), lambda b: (0, b)),
        ],
        out_shape=[
            jax.ShapeDtypeStruct((ROWS, TC_GRID * TW), jnp.float32),
            jax.ShapeDtypeStruct((ROWS, TC_GRID * TW), jnp.int32),
        ],
        compiler_params=pltpu.CompilerParams(
            dimension_semantics=("parallel",),
        ),
    )(inputs)
    out = pl.pallas_call(
        _merge_kernel,
        out_shape=jax.ShapeDtypeStruct((ROWS, 1), jnp.int32),
    )(tc_vals, tc_idxs, sc_vals.reshape(ROWS, 16), sc_idxs.reshape(ROWS, 16),
      inputs[:, SLIVER_START:])
    return out.reshape(ROWS)


def kernel(inputs, states):
    predicted_ids = _sample(inputs)
    return (predicted_ids, states)
